# baseline (speedup 1.0000x reference)
import functools

import jax
import jax.numpy as jnp
from jax import lax
from jax.experimental import pallas as pl
from jax.experimental.pallas import tpu as pltpu
from jax.experimental.pallas import tpu_sc as plsc

N = 10000
E = 320000
D = 128
K = 4
H = 128

NC = 2
NS = 16
NW = NC * NS
L = 16

EC = 64
CPP = 320
N0 = 80
N1 = CPP - N0
TOTC = NS * CPP
EPAD = TOTC * EC
BLK = 8
GC = 128
GPT = -(-(N * K // NW) // GC)
GPAD = NW * GPT * GC
DEPT = 96
DEC = 112
DPAD = NW * DEPT * DEC

NPAD = 10240
RPS = NPAD // NS

_mesh = plsc.VectorSubcoreMesh(core_axis_name="c", subcore_axis_name="s")
_sc_params = pltpu.CompilerParams(needs_layout_passes=False)


@functools.partial(
    pl.kernel,
    out_type=(
        jax.ShapeDtypeStruct((GPAD, D), jnp.float32),
        jax.ShapeDtypeStruct((NC, NPAD), jnp.float32),
    ),
    mesh=_mesh,
    scratch_types=[
        pltpu.VMEM((N,), jnp.int32),
        pltpu.VMEM((GPT, GC), jnp.int32),
        pltpu.VMEM((GPT, GC), jnp.int32),
        pltpu.VMEM((GC, D), jnp.float32),
        pltpu.VMEM((GC, D), jnp.float32),
        pltpu.VMEM((DEPT, DEC), jnp.int32),
        pltpu.VMEM((DEPT, DEC), jnp.float32),
        pltpu.VMEM_SHARED((NPAD,), jnp.float32),
        pltpu.SemaphoreType.DMA,
        pltpu.SemaphoreType.DMA,
        pltpu.SemaphoreType.DMA,
    ],
    compiler_params=_sc_params,
)
def _sc_gather_deg(nodes_hbm, nbr_hbm, dst_hbm, w_hbm, zeros1_hbm, emb_hbm,
                   comb_hbm, degp_hbm,
                   nodes_v, nb_v, idx_v, rows_a, rows_b, dst_v, w_v, deg_sh,
                   gsem_a, gsem_b, dsem):
    c = lax.axis_index("c")
    s = lax.axis_index("s")
    wid = s * NC + c

    pltpu.sync_copy(zeros1_hbm.at[pl.ds(s * RPS, RPS)],
                    deg_sh.at[pl.ds(s * RPS, RPS)])

    pltpu.sync_copy(nodes_hbm, nodes_v)
    pltpu.sync_copy(nbr_hbm.at[wid], nb_v)
    pltpu.sync_copy(dst_hbm.at[wid], dst_v)
    pltpu.sync_copy(w_hbm.at[wid], w_v)
    plsc.subcore_barrier()

    def _deg(j, _):
        pltpu.async_copy(w_v.at[j], deg_sh.at[dst_v.at[j]], dsem, add=True)
        return 0
    lax.fori_loop(0, DEPT, _deg, 0)

    def _xlate(j, _):
        def _grp(g, _):
            nb16 = nb_v[j, pl.ds(g * L, L)]
            idx_v[j, pl.ds(g * L, L)] = plsc.load_gather(nodes_v, [nb16])
            return 0
        lax.fori_loop(0, GC // L, _grp, 0)
        return 0
    lax.fori_loop(0, GPT, _xlate, 0)

    rows = (rows_a, rows_b)
    gsems = (gsem_a, gsem_b)
    pltpu.async_copy(emb_hbm.at[idx_v.at[0]], rows_a, gsem_a)
    pltpu.async_copy(emb_hbm.at[idx_v.at[1]], rows_b, gsem_b)

    def _rows(p, _):
        for b in range(2):
            j = 2 * p + b
            pltpu.make_async_copy(emb_hbm.at[idx_v.at[j]], rows[b],
                                  gsems[b]).wait()
            pltpu.sync_copy(rows[b],
                            comb_hbm.at[pl.ds((wid * GPT + j) * GC, GC)])

            @pl.when(p < GPT // 2 - 1)
            def _():
                pltpu.async_copy(emb_hbm.at[idx_v.at[j + 2]], rows[b],
                                 gsems[b])
        return 0
    lax.fori_loop(0, GPT // 2, _rows, 0)

    def _drain(j, _):
        pltpu.make_async_copy(w_v.at[j], deg_sh.at[dst_v.at[j]], dsem).wait()
        return 0
    lax.fori_loop(0, DEPT, _drain, 0)
    plsc.subcore_barrier()
    pltpu.sync_copy(deg_sh.at[pl.ds(s * RPS, RPS)],
                    degp_hbm.at[c, pl.ds(s * RPS, RPS)])


@functools.partial(
    pl.kernel,
    out_type=jax.ShapeDtypeStruct((NC, NPAD, D), jnp.float32),
    mesh=_mesh,
    scratch_types=[
        pltpu.VMEM((BLK, EC), jnp.int32),
        pltpu.VMEM((BLK, EC), jnp.int32),
        pltpu.VMEM((BLK, EC), jnp.int32),
        pltpu.VMEM((BLK, EC), jnp.int32),
        pltpu.VMEM((BLK, EC), jnp.float32),
        pltpu.VMEM((BLK, EC), jnp.float32),
        pltpu.VMEM((EC, D), jnp.float32),
        pltpu.VMEM((EC, D), jnp.float32),
        pltpu.VMEM((EC, D), jnp.float32),
        pltpu.VMEM((EC, D), jnp.float32),
        pltpu.VMEM_SHARED((NPAD, D), jnp.float32),
        pltpu.SemaphoreType.DMA,
        pltpu.SemaphoreType.DMA,
        pltpu.SemaphoreType.DMA,
        pltpu.SemaphoreType.DMA,
        pltpu.SemaphoreType.DMA,
    ],
    compiler_params=_sc_params,
)
def _sc_mp(hs_hbm, src_hbm, dst_hbm, w_hbm, zeros2_hbm, part_hbm,
           srcb0, srcb1, dstb0, dstb1, wb0, wb1,
           gb_a, gb_b, sb_a, sb_b, agg_sh,
           gsem_a, gsem_b, ssem_a, ssem_b, bsem):
    c = lax.axis_index("c")
    s = lax.axis_index("s")

    nn = jnp.where(c == 0, N0, N1)
    base = jnp.where(c == 0, s * N0, NS * N0 + s * N1)
    npairs = nn // (2 * BLK)

    srcb = (srcb0, srcb1)
    dstb = (dstb0, dstb1)
    wb = (wb0, wb1)
    gbufs = (gb_a, gb_b)
    sbufs = (sb_a, sb_b)
    gsems = (gsem_a, gsem_b)
    ssems = (ssem_a, ssem_b)

    pltpu.sync_copy(zeros2_hbm.at[pl.ds(s * RPS, RPS)],
                    agg_sh.at[pl.ds(s * RPS, RPS)])
    for t in range(2):
        pltpu.sync_copy(src_hbm.at[pl.ds(base + t * BLK, BLK)], srcb[t])
        pltpu.sync_copy(dst_hbm.at[pl.ds(base + t * BLK, BLK)], dstb[t])
        pltpu.sync_copy(w_hbm.at[pl.ds(base + t * BLK, BLK)], wb[t])
    pltpu.async_copy(hs_hbm.at[srcb0.at[0]], gb_a, gsem_a)
    pltpu.async_copy(hs_hbm.at[srcb0.at[1]], gb_b, gsem_b)
    plsc.subcore_barrier()

    def _scale(t, k, b):
        def _grp(g, _):
            w16 = wb[t][k, pl.ds(g * L, L)]
            for i in range(L):
                spl = lax.broadcast(w16[i], (L,))
                e = g * L + i
                for cb in range(D // L):
                    sbufs[b][e, pl.ds(cb * L, L)] = (
                        gbufs[b][e, pl.ds(cb * L, L)] * spl)
            return 0
        lax.fori_loop(0, EC // L, _grp, 0)

    def _refill(p, t):
        nb_ds = pl.ds(base + (2 * p + t + 1) * BLK, BLK)
        pltpu.async_copy(src_hbm.at[nb_ds], srcb[1 - t], bsem)
        pltpu.async_copy(dst_hbm.at[nb_ds], dstb[1 - t], bsem)
        pltpu.async_copy(w_hbm.at[nb_ds], wb[1 - t], bsem)

    def _wait_refill(p, t):
        nb_ds = pl.ds(base + (2 * p + t + 1) * BLK, BLK)
        pltpu.make_async_copy(src_hbm.at[nb_ds], srcb[1 - t], bsem).wait()
        pltpu.make_async_copy(dst_hbm.at[nb_ds], dstb[1 - t], bsem).wait()
        pltpu.make_async_copy(w_hbm.at[nb_ds], wb[1 - t], bsem).wait()

    def _blockpair(p, _):
        for t in range(2):
            nxt = 1 - t
            has_refill = (p >= 1) if t == 0 else (p < npairs - 1)

            def _qstep(q, _, t=t, nxt=nxt, has_refill=has_refill):
                for i in range(2):
                    k = 2 * q + i
                    pltpu.make_async_copy(hs_hbm.at[srcb[t].at[k]],
                                          gbufs[i], gsems[i]).wait()
                    if t == 0:
                        @pl.when((p >= 1) | (q >= 1))
                        def _():
                            pltpu.make_async_copy(
                                sbufs[i], agg_sh.at[dstb[t].at[k]],
                                ssems[i]).wait()
                    else:
                        pltpu.make_async_copy(
                            sbufs[i], agg_sh.at[dstb[t].at[k]],
                            ssems[i]).wait()

                    _scale(t, k, i)

                    pltpu.async_copy(sbufs[i], agg_sh.at[dstb[t].at[k]],
                                     ssems[i], add=True)

                    if i == 0:
                        @pl.when((q == 1) & has_refill)
                        def _():
                            _refill(p, t)

                    @pl.when(q < BLK // 2 - 1)
                    def _():
                        pltpu.async_copy(hs_hbm.at[srcb[t].at[k + 2]],
                                         gbufs[i], gsems[i])

                    @pl.when(q == BLK // 2 - 1)
                    def _():
                        if i == 0:
                            @pl.when(has_refill)
                            def _():
                                _wait_refill(p, t)
                        if t == 0:
                            pltpu.async_copy(hs_hbm.at[srcb[nxt].at[i]],
                                             gbufs[i], gsems[i])
                        else:
                            @pl.when(has_refill)
                            def _():
                                pltpu.async_copy(hs_hbm.at[srcb[nxt].at[i]],
                                                 gbufs[i], gsems[i])
                return 0
            lax.fori_loop(0, BLK // 2, _qstep, 0)
        return 0
    lax.fori_loop(0, npairs, _blockpair, 0)

    for b in range(2):
        pltpu.make_async_copy(sbufs[b], agg_sh.at[dstb[1].at[BLK - 2 + b]],
                              ssems[b]).wait()

    plsc.subcore_barrier()
    pltpu.sync_copy(agg_sh.at[pl.ds(s * RPS, RPS)],
                    part_hbm.at[c, pl.ds(s * RPS, RPS)])


_R = 1024


def _tc1_body(comb_ref, degp_ref, w1_ref, out_ref):
    deg = degp_ref[0, :] + degp_ref[1, :] + 1.0
    dinv = lax.rsqrt(deg)
    h = jnp.dot(comb_ref[...], w1_ref[...], preferred_element_type=jnp.float32)
    out_ref[...] = h * dinv[:, None]


def _tc1(comb, degp, w1):
    return pl.pallas_call(
        _tc1_body,
        grid=(NPAD // _R,),
        in_specs=[
            pl.BlockSpec((_R, K * D), lambda i: (i, 0)),
            pl.BlockSpec((NC, _R), lambda i: (0, i)),
            pl.BlockSpec((K * D, H), lambda i: (0, 0)),
        ],
        out_specs=pl.BlockSpec((_R, H), lambda i: (i, 0)),
        out_shape=jax.ShapeDtypeStruct((NPAD, H), jnp.float32),
    )(comb, degp, w1)


def _tc2_body(part_ref, hs_ref, degp_ref, b1_ref, w2_ref, out_ref):
    deg = degp_ref[0, :] + degp_ref[1, :] + 1.0
    dinv = lax.rsqrt(deg)
    p = part_ref[0] + part_ref[1] + hs_ref[...]
    x2 = jnp.maximum(p * dinv[:, None] + b1_ref[...], 0.0)
    h = jnp.dot(x2, w2_ref[...], preferred_element_type=jnp.float32)
    out_ref[...] = h * dinv[:, None]


def _tc2(part, hs, degp, b1, w2):
    return pl.pallas_call(
        _tc2_body,
        grid=(NPAD // _R,),
        in_specs=[
            pl.BlockSpec((NC, _R, H), lambda i: (0, i, 0)),
            pl.BlockSpec((_R, H), lambda i: (i, 0)),
            pl.BlockSpec((NC, _R), lambda i: (0, i)),
            pl.BlockSpec((1, H), lambda i: (0, 0)),
            pl.BlockSpec((H, D), lambda i: (0, 0)),
        ],
        out_specs=pl.BlockSpec((_R, D), lambda i: (i, 0)),
        out_shape=jax.ShapeDtypeStruct((NPAD, D), jnp.float32),
    )(part, hs, degp, b1, w2)


def _tc3_body(part_ref, hs_ref, degp_ref, b2_ref, out_ref):
    deg = degp_ref[0, :] + degp_ref[1, :] + 1.0
    dinv = lax.rsqrt(deg)
    p = part_ref[0] + part_ref[1] + hs_ref[...]
    out_ref[...] = p * dinv[:, None] + b2_ref[...]


def _tc3(part, hs, degp, b2):
    return pl.pallas_call(
        _tc3_body,
        grid=(NPAD // _R,),
        in_specs=[
            pl.BlockSpec((NC, _R, D), lambda i: (0, i, 0)),
            pl.BlockSpec((_R, D), lambda i: (i, 0)),
            pl.BlockSpec((NC, _R), lambda i: (0, i)),
            pl.BlockSpec((1, D), lambda i: (0, 0)),
        ],
        out_specs=pl.BlockSpec((_R, D), lambda i: (i, 0)),
        out_shape=jax.ShapeDtypeStruct((NPAD, D), jnp.float32),
    )(part, hs, degp, b2)


def kernel(nodes, edge_index, edge_weights, neighbor_idx, emb_table,
           W1, b1, W2, b2):
    i32 = jnp.int32
    src = jnp.concatenate(
        [edge_index[0], jnp.zeros((EPAD - E,), i32)]).reshape(TOTC, EC)
    dst = jnp.concatenate(
        [edge_index[1], jnp.zeros((EPAD - E,), i32)]).reshape(TOTC, EC)
    w = jnp.concatenate(
        [edge_weights, jnp.zeros((EPAD - E,), jnp.float32)]
    ).reshape(TOTC, EC)
    dstd = jnp.concatenate(
        [edge_index[1], jnp.zeros((DPAD - E,), i32)]).reshape(NW, DEPT, DEC)
    wd = jnp.concatenate(
        [edge_weights, jnp.zeros((DPAD - E,), jnp.float32)]
    ).reshape(NW, DEPT, DEC)
    nbr = jnp.concatenate(
        [neighbor_idx.reshape(-1), jnp.zeros((GPAD - N * K,), i32)]
    ).reshape(NW, GPT, GC)
    zeros1 = jnp.zeros((NPAD,), jnp.float32)
    zeros2 = jnp.zeros((NPAD, D), jnp.float32)

    comb_rows, degp = _sc_gather_deg(nodes, nbr, dstd, wd, zeros1, emb_table)
    comb = comb_rows.reshape(NPAD, K * D)

    hs1 = _tc1(comb, degp, W1)
    part1 = _sc_mp(hs1, src, dst, w, zeros2)
    hs2 = _tc2(part1, hs1, degp, b1.reshape(1, H), W2)
    part2 = _sc_mp(hs2, src, dst, w, zeros2)
    out = _tc3(part2, hs2, degp, b2.reshape(1, D))
    return out[:N]


# speedup vs baseline: 1.4571x; 1.4571x over previous
import functools

import jax
import jax.numpy as jnp
from jax import lax
from jax.experimental import pallas as pl
from jax.experimental.pallas import tpu as pltpu
from jax.experimental.pallas import tpu_sc as plsc

N = 10000
E = 320000
D = 128
K = 4
H = 128

NC = 2
NS = 16
NW = NC * NS
L = 16

EC = 128
EPT = -(-(E // NW) // EC)
EPAD = NW * EPT * EC
GC = 128
GPT = -(-(N * K // NW) // GC)
GPAD = NW * GPT * GC
DEPT = 96
DEC = 112
DPAD = NW * DEPT * DEC

NPAD = 10240
RPS = NPAD // NS

_mesh = plsc.VectorSubcoreMesh(core_axis_name="c", subcore_axis_name="s")
_sc_params = pltpu.CompilerParams(needs_layout_passes=False)


@functools.partial(
    pl.kernel,
    out_type=(
        jax.ShapeDtypeStruct((GPAD, D), jnp.float32),
        jax.ShapeDtypeStruct((NC, NPAD), jnp.float32),
    ),
    mesh=_mesh,
    scratch_types=[
        pltpu.VMEM((N,), jnp.int32),
        pltpu.VMEM((GPT, GC), jnp.int32),
        pltpu.VMEM((GPT, GC), jnp.int32),
        pltpu.VMEM((GC, D), jnp.float32),
        pltpu.VMEM((GC, D), jnp.float32),
        pltpu.VMEM((DEPT, DEC), jnp.int32),
        pltpu.VMEM((DEPT, DEC), jnp.float32),
        pltpu.VMEM_SHARED((NPAD,), jnp.float32),
        pltpu.SemaphoreType.DMA,
        pltpu.SemaphoreType.DMA,
        pltpu.SemaphoreType.DMA,
    ],
    compiler_params=_sc_params,
)
def _sc_gather_deg(nodes_hbm, nbr_hbm, dst_hbm, w_hbm, zeros1_hbm, emb_hbm,
                   comb_hbm, degp_hbm,
                   nodes_v, nb_v, idx_v, rows_a, rows_b, dst_v, w_v, deg_sh,
                   gsem_a, gsem_b, dsem):
    c = lax.axis_index("c")
    s = lax.axis_index("s")
    wid = s * NC + c

    pltpu.sync_copy(zeros1_hbm.at[pl.ds(s * RPS, RPS)],
                    deg_sh.at[pl.ds(s * RPS, RPS)])

    pltpu.sync_copy(nodes_hbm, nodes_v)
    pltpu.sync_copy(nbr_hbm.at[wid], nb_v)
    pltpu.sync_copy(dst_hbm.at[wid], dst_v)
    pltpu.sync_copy(w_hbm.at[wid], w_v)
    plsc.subcore_barrier()

    def _deg(j, _):
        pltpu.async_copy(w_v.at[j], deg_sh.at[dst_v.at[j]], dsem, add=True)
        return 0
    lax.fori_loop(0, DEPT, _deg, 0)

    def _xlate(j, _):
        def _grp(g, _):
            nb16 = nb_v[j, pl.ds(g * L, L)]
            idx_v[j, pl.ds(g * L, L)] = plsc.load_gather(nodes_v, [nb16])
            return 0
        lax.fori_loop(0, GC // L, _grp, 0)
        return 0
    lax.fori_loop(0, GPT, _xlate, 0)

    rows = (rows_a, rows_b)
    gsems = (gsem_a, gsem_b)
    pltpu.async_copy(emb_hbm.at[idx_v.at[0]], rows_a, gsem_a)
    pltpu.async_copy(emb_hbm.at[idx_v.at[1]], rows_b, gsem_b)

    def _rows(p, _):
        for b in range(2):
            j = 2 * p + b
            pltpu.make_async_copy(emb_hbm.at[idx_v.at[j]], rows[b],
                                  gsems[b]).wait()
            pltpu.sync_copy(rows[b],
                            comb_hbm.at[pl.ds((wid * GPT + j) * GC, GC)])

            @pl.when(p < GPT // 2 - 1)
            def _():
                pltpu.async_copy(emb_hbm.at[idx_v.at[j + 2]], rows[b],
                                 gsems[b])
        return 0
    lax.fori_loop(0, GPT // 2, _rows, 0)

    def _drain(j, _):
        pltpu.make_async_copy(w_v.at[j], deg_sh.at[dst_v.at[j]], dsem).wait()
        return 0
    lax.fori_loop(0, DEPT, _drain, 0)
    plsc.subcore_barrier()
    pltpu.sync_copy(deg_sh.at[pl.ds(s * RPS, RPS)],
                    degp_hbm.at[c, pl.ds(s * RPS, RPS)])


@functools.partial(
    pl.kernel,
    out_type=jax.ShapeDtypeStruct((NC, NPAD, D), jnp.float32),
    mesh=_mesh,
    scratch_types=[
        pltpu.VMEM((EPT, EC), jnp.int32),
        pltpu.VMEM((EPT, EC), jnp.int32),
        pltpu.VMEM((EPT, EC), jnp.float32),
        pltpu.VMEM((EC, D), jnp.float32),
        pltpu.VMEM_SHARED((NPAD, D), jnp.float32),
        pltpu.SemaphoreType.DMA,
    ],
    compiler_params=_sc_params,
)
def _sc_mp(hs_hbm, src_hbm, dst_hbm, w_hbm, zeros2_hbm, part_hbm,
           src_v, dst_v, w_v, rows_v, agg_sh, sem):
    c = lax.axis_index("c")
    s = lax.axis_index("s")
    wid = s * NC + c

    pltpu.sync_copy(zeros2_hbm.at[pl.ds(s * RPS, RPS)],
                    agg_sh.at[pl.ds(s * RPS, RPS)])
    pltpu.sync_copy(src_hbm.at[wid], src_v)
    pltpu.sync_copy(dst_hbm.at[wid], dst_v)
    pltpu.sync_copy(w_hbm.at[wid], w_v)
    plsc.subcore_barrier()

    def _chunk(j, _):
        pltpu.async_copy(hs_hbm.at[src_v.at[j]], rows_v, sem).wait()

        def _grp(g, _):
            w16 = w_v[j, pl.ds(g * L, L)]
            for i in range(L):
                spl = lax.broadcast(w16[i], (L,))
                e = g * L + i
                for cb in range(D // L):
                    rows_v[e, pl.ds(cb * L, L)] = (
                        rows_v[e, pl.ds(cb * L, L)] * spl)
            return 0
        lax.fori_loop(0, EC // L, _grp, 0)

        pltpu.sync_copy(rows_v, agg_sh.at[dst_v.at[j]], add=True)
        return 0
    lax.fori_loop(0, EPT, _chunk, 0)

    plsc.subcore_barrier()
    pltpu.sync_copy(agg_sh.at[pl.ds(s * RPS, RPS)],
                    part_hbm.at[c, pl.ds(s * RPS, RPS)])


_R = 1024


def _tc1_body(comb_ref, degp_ref, w1_ref, out_ref):
    deg = degp_ref[0, :] + degp_ref[1, :] + 1.0
    dinv = lax.rsqrt(deg)
    h = jnp.dot(comb_ref[...], w1_ref[...], preferred_element_type=jnp.float32)
    out_ref[...] = h * dinv[:, None]


def _tc1(comb, degp, w1):
    return pl.pallas_call(
        _tc1_body,
        grid=(NPAD // _R,),
        in_specs=[
            pl.BlockSpec((_R, K * D), lambda i: (i, 0)),
            pl.BlockSpec((NC, _R), lambda i: (0, i)),
            pl.BlockSpec((K * D, H), lambda i: (0, 0)),
        ],
        out_specs=pl.BlockSpec((_R, H), lambda i: (i, 0)),
        out_shape=jax.ShapeDtypeStruct((NPAD, H), jnp.float32),
    )(comb, degp, w1)


def _tc2_body(part_ref, hs_ref, degp_ref, b1_ref, w2_ref, out_ref):
    deg = degp_ref[0, :] + degp_ref[1, :] + 1.0
    dinv = lax.rsqrt(deg)
    p = part_ref[0] + part_ref[1] + hs_ref[...]
    x2 = jnp.maximum(p * dinv[:, None] + b1_ref[...], 0.0)
    h = jnp.dot(x2, w2_ref[...], preferred_element_type=jnp.float32)
    out_ref[...] = h * dinv[:, None]


def _tc2(part, hs, degp, b1, w2):
    return pl.pallas_call(
        _tc2_body,
        grid=(NPAD // _R,),
        in_specs=[
            pl.BlockSpec((NC, _R, H), lambda i: (0, i, 0)),
            pl.BlockSpec((_R, H), lambda i: (i, 0)),
            pl.BlockSpec((NC, _R), lambda i: (0, i)),
            pl.BlockSpec((1, H), lambda i: (0, 0)),
            pl.BlockSpec((H, D), lambda i: (0, 0)),
        ],
        out_specs=pl.BlockSpec((_R, D), lambda i: (i, 0)),
        out_shape=jax.ShapeDtypeStruct((NPAD, D), jnp.float32),
    )(part, hs, degp, b1, w2)


def _tc3_body(part_ref, hs_ref, degp_ref, b2_ref, out_ref):
    deg = degp_ref[0, :] + degp_ref[1, :] + 1.0
    dinv = lax.rsqrt(deg)
    p = part_ref[0] + part_ref[1] + hs_ref[...]
    out_ref[...] = p * dinv[:, None] + b2_ref[...]


def _tc3(part, hs, degp, b2):
    return pl.pallas_call(
        _tc3_body,
        grid=(NPAD // _R,),
        in_specs=[
            pl.BlockSpec((NC, _R, D), lambda i: (0, i, 0)),
            pl.BlockSpec((_R, D), lambda i: (i, 0)),
            pl.BlockSpec((NC, _R), lambda i: (0, i)),
            pl.BlockSpec((1, D), lambda i: (0, 0)),
        ],
        out_specs=pl.BlockSpec((_R, D), lambda i: (i, 0)),
        out_shape=jax.ShapeDtypeStruct((NPAD, D), jnp.float32),
    )(part, hs, degp, b2)


def kernel(nodes, edge_index, edge_weights, neighbor_idx, emb_table,
           W1, b1, W2, b2):
    i32 = jnp.int32
    src = jnp.concatenate(
        [edge_index[0], jnp.zeros((EPAD - E,), i32)]).reshape(NW, EPT, EC)
    dst = jnp.concatenate(
        [edge_index[1], jnp.zeros((EPAD - E,), i32)]).reshape(NW, EPT, EC)
    w = jnp.concatenate(
        [edge_weights, jnp.zeros((EPAD - E,), jnp.float32)]
    ).reshape(NW, EPT, EC)
    dstd = jnp.concatenate(
        [edge_index[1], jnp.zeros((DPAD - E,), i32)]).reshape(NW, DEPT, DEC)
    wd = jnp.concatenate(
        [edge_weights, jnp.zeros((DPAD - E,), jnp.float32)]
    ).reshape(NW, DEPT, DEC)
    nbr = jnp.concatenate(
        [neighbor_idx.reshape(-1), jnp.zeros((GPAD - N * K,), i32)]
    ).reshape(NW, GPT, GC)
    zeros1 = jnp.zeros((NPAD,), jnp.float32)
    zeros2 = jnp.zeros((NPAD, D), jnp.float32)

    comb_rows, degp = _sc_gather_deg(nodes, nbr, dstd, wd, zeros1, emb_table)
    comb = comb_rows.reshape(NPAD, K * D)

    hs1 = _tc1(comb, degp, W1)
    part1 = _sc_mp(hs1, src, dst, w, zeros2)
    hs2 = _tc2(part1, hs1, degp, b1.reshape(1, H), W2)
    part2 = _sc_mp(hs2, src, dst, w, zeros2)
    out = _tc3(part2, hs2, degp, b2.reshape(1, D))
    return out[:N]
